# baseline (speedup 1.0000x reference)
import functools

import jax
import jax.numpy as jnp
from jax import lax
from jax.experimental import pallas as pl
from jax.experimental.pallas import tpu as pltpu
from jax.experimental.pallas import tpu_sc as plsc

_EPS = 1e-5
_INTERPRET = False


def _dot(a, b, ca, cb, precision=None):
    return jax.lax.dot_general(
        a, b, (((ca,), (cb,)), ((), ())),
        preferred_element_type=jnp.float32, precision=precision)


def _knn_body(k, kp, orig_ref, q_ref, rel_ref, idxg_ref, srel_ref,
              srelsq_ref, m_const):
    b = pl.program_id(0)
    n = pl.program_id(1)
    orig = orig_ref[0]
    q = q_ref[0]
    tn = q.shape[1]

    cross = _dot(orig, q, 0, 0)
    q2 = jnp.sum(q * q, axis=0, keepdims=True)
    o2row = jnp.sum(orig * orig, axis=0, keepdims=True)
    o2col = jnp.swapaxes(o2row, 0, 1)
    dist = (q2 - 2.0 * cross) + o2col

    srel = jnp.zeros((3, 1), jnp.float32)
    srelsq = jnp.zeros((3, 3), jnp.float32)
    hp = jax.lax.Precision.HIGHEST

    def split3(x):
        bf = jnp.bfloat16
        hi = x.astype(bf)
        r1 = x - hi.astype(jnp.float32)
        mid = r1.astype(bf)
        lo = (r1 - mid.astype(jnp.float32)).astype(bf)
        return hi, mid, lo

    def oh_dot(parts, ohb):
        acc = None
        for p in parts:
            t = _dot(p, ohb, 1, 0)
            acc = t if acc is None else acc + t
        return acc

    orig3 = split3(orig)
    half = m_const // 2
    da = dist[:half]
    db = dist[half:]
    iotap = jax.lax.broadcasted_iota(jnp.int32, (half, tn), 0)
    leq = da <= db
    pmin = jnp.where(leq, da, db)
    pmax = jnp.where(leq, db, da)
    pidx = jnp.where(leq, iotap, iotap + half)
    oas, obs = [], []
    oriap = tuple(p[:, :half] for p in orig3)
    oribp = tuple(p[:, half:] for p in orig3)
    idxs = []
    inf = jnp.float32(jnp.inf)
    for j in range(k):
        m = jnp.min(pmin, axis=0, keepdims=True)
        idx = jnp.min(jnp.where(pmin == m, pidx, m_const), axis=0,
                      keepdims=True)
        hit = pidx == idx
        lowsel = idx < half
        oas.append((hit & lowsel).astype(jnp.bfloat16))
        obs.append((hit & (~lowsel)).astype(jnp.bfloat16))
        pmin = jnp.where(hit, pmax, pmin)
        pmax = jnp.where(hit, inf, pmax)
        pidx = jnp.where(hit, pidx ^ half, pidx)
        if j < kp:
            idxs.append(idx)
            if j == kp - 1:
                idxg_ref[0, 0] = jnp.concatenate(idxs, axis=0) + b * m_const
        if (j + 1) % 4 == 0:
            ohcat_a = jnp.concatenate(oas, axis=1)
            ohcat_b = jnp.concatenate(obs, axis=1)
            oas, obs = [], []
            ptsc = oh_dot(oriap, ohcat_a) + oh_dot(oribp, ohcat_b)
            for jj in range(4):
                jr = j - 3 + jj
                rel = ptsc[:, jj * tn:(jj + 1) * tn] - q
                rel_ref[0, jr] = rel
                srel = srel + jnp.sum(rel, axis=1, keepdims=True)
                srelsq = srelsq + _dot(rel, rel, 1, 1, precision=hp)

    first = (b == 0) & (n == 0)

    @pl.when(first)
    def _():
        srel_ref[...] = jnp.zeros_like(srel_ref)
        srelsq_ref[...] = jnp.zeros_like(srelsq_ref)

    srel_ref[...] += srel
    srelsq_ref[...] += srelsq


def _mid_body(k, kp, cnt, rel_ref, srel_ref, srelsq_ref, w0_ref, g0_ref,
              be0_ref, w1_ref, y14_ref, fg_ref, ssum1_ref, ssq1_ref):
    b = pl.program_id(0)
    n = pl.program_id(1)
    w0 = w0_ref[...]
    w1 = w1_ref[...]
    mean_rel = srel_ref[...] / cnt
    cov = (srelsq_ref[...] / cnt
           - _dot(mean_rel, mean_rel, 1, 1))
    mean0 = _dot(w0, mean_rel, 1, 0)
    wc = _dot(w0, cov, 1, 0)
    var0 = jnp.sum(wc * w0, axis=1, keepdims=True)
    scale0 = g0_ref[...] * jax.lax.rsqrt(var0 + _EPS)
    shift0 = be0_ref[...] - mean0 * scale0

    fg = None
    ssum1 = jnp.zeros_like(ssum1_ref)
    ssq1 = jnp.zeros_like(ssq1_ref)
    for j in range(k):
        y0 = _dot(w0, rel_ref[0, j], 1, 0)
        f0 = jax.nn.relu(y0 * scale0 + shift0)
        fg = f0 if fg is None else jnp.maximum(fg, f0)
        y1 = _dot(w1, f0, 1, 0)
        ssum1 = ssum1 + jnp.sum(y1, axis=1, keepdims=True)
        ssq1 = ssq1 + jnp.sum(y1 * y1, axis=1, keepdims=True)
        if j < kp:
            y14_ref[0, :, j] = y1
    fg_ref[0] = fg

    first = (b == 0) & (n == 0)

    @pl.when(first)
    def _():
        ssum1_ref[...] = jnp.zeros_like(ssum1_ref)
        ssq1_ref[...] = jnp.zeros_like(ssq1_ref)

    ssum1_ref[...] += ssum1
    ssq1_ref[...] += ssq1


def _sc_gather(table, gidx):
    tot = gidx.shape[0]
    d = table.shape[1]
    info = plsc.get_sparse_core_info()
    nc, ns = info.num_cores, info.num_subcores
    nw = nc * ns
    per = tot // nw
    ch = 128
    mesh = plsc.VectorSubcoreMesh(core_axis_name="c", subcore_axis_name="s")

    @functools.partial(
        pl.kernel, mesh=mesh,
        out_type=jax.ShapeDtypeStruct((tot, d), jnp.float32),
        scratch_types=[
            pltpu.VMEM((ch,), jnp.int32),
            pltpu.VMEM((ch, d), jnp.float32),
            pltpu.SemaphoreType.DMA,
        ],
    )
    def k(table_hbm, idx_hbm, out_hbm, idx_v, rows_v, sem):
        wid = lax.axis_index("s") * nc + lax.axis_index("c")
        base = wid * per
        for c in range(per // ch):
            off = base + c * ch
            pltpu.sync_copy(idx_hbm.at[pl.ds(off, ch)], idx_v)
            pltpu.async_copy(table_hbm.at[idx_v], rows_v, sem).wait()
            pltpu.sync_copy(rows_v, out_hbm.at[pl.ds(off, ch)])

    return k(table, gidx)


def _out_body(kp, cnt, y14_ref, fg_ref, patch_ref, ssum1_ref, ssq1_ref,
              g1_ref, be1_ref, w2a_ref, w2b_ref, b2_ref, out_ref):
    mean1 = ssum1_ref[...] / cnt
    var1 = ssq1_ref[...] / cnt - mean1 * mean1
    scale1 = g1_ref[...] * jax.lax.rsqrt(var1 + _EPS)
    shift1 = be1_ref[...] - mean1 * scale1

    fg = fg_ref[0]
    fterm = _dot(w2b_ref[...], fg, 1, 0) + b2_ref[...]
    acc = None
    for j in range(kp):
        rf = jax.nn.relu(y14_ref[0, :, j] * scale1 + shift1)
        logit = _dot(w2a_ref[...], rf, 1, 0) + fterm
        w = jax.nn.sigmoid(logit)
        pj = jnp.swapaxes(patch_ref[0, j], 0, 1)
        term = (1.0 - w) * rf + w * pj
        acc = term if acc is None else acc + term
    out_ref[0] = acc


def kernel(original_pts, query_pts, local_feat, w0, b0, g0, be0, w1, b1, g1,
           be1, w2, b2):
    del b0, b1
    B, _, M = original_pts.shape
    N = query_pts.shape[2]
    D = local_feat.shape[1]
    K = 16
    KP = K // 4
    TN = 512 if N % 512 == 0 else N
    NT = N // TN
    cnt = float(B * N * K)
    f32 = jnp.float32

    g0c = g0.reshape(D, 1)
    be0c = be0.reshape(D, 1)
    g1c = g1.reshape(D, 1)
    be1c = be1.reshape(D, 1)
    w2a = w2[0, :D].reshape(1, D)
    w2b = w2[0, D:].reshape(1, D)
    b2c = b2.reshape(1, 1)

    const2 = lambda shape: pl.BlockSpec(shape, lambda b, n: (0,) * len(shape))

    rel, idx4, srel, srelsq = pl.pallas_call(
        functools.partial(_knn_body, K, KP, m_const=M),
        grid=(B, NT),
        in_specs=[
            pl.BlockSpec((1, 3, M), lambda b, n: (b, 0, 0)),
            pl.BlockSpec((1, 3, TN), lambda b, n: (b, 0, n)),
        ],
        out_specs=[
            pl.BlockSpec((1, K, 3, TN), lambda b, n: (b, 0, 0, n)),
            pl.BlockSpec((1, 1, KP, TN), lambda b, n: (b, n, 0, 0)),
            const2((3, 1)),
            const2((3, 3)),
        ],
        out_shape=[
            jax.ShapeDtypeStruct((B, K, 3, N), f32),
            jax.ShapeDtypeStruct((B, NT, KP, TN), jnp.int32),
            jax.ShapeDtypeStruct((3, 1), f32),
            jax.ShapeDtypeStruct((3, 3), f32),
        ],
        interpret=_INTERPRET,
    )(original_pts, query_pts)

    table = jnp.swapaxes(local_feat, 1, 2).reshape(B * M, D)
    rows = _sc_gather(table, idx4.reshape(B * NT * KP * TN))
    patch = rows.reshape(B * NT, KP, TN, D)

    y14, fg, ssum1, ssq1 = pl.pallas_call(
        functools.partial(_mid_body, K, KP, cnt),
        grid=(B, NT),
        in_specs=[
            pl.BlockSpec((1, K, 3, TN), lambda b, n: (b, 0, 0, n)),
            const2((3, 1)),
            const2((3, 3)),
            const2((D, 3)),
            const2((D, 1)),
            const2((D, 1)),
            const2((D, D)),
        ],
        out_specs=[
            pl.BlockSpec((1, D, KP, TN), lambda b, n: (b, 0, 0, n)),
            pl.BlockSpec((1, D, TN), lambda b, n: (b, 0, n)),
            const2((D, 1)),
            const2((D, 1)),
        ],
        out_shape=[
            jax.ShapeDtypeStruct((B, D, KP, N), f32),
            jax.ShapeDtypeStruct((B, D, N), f32),
            jax.ShapeDtypeStruct((D, 1), f32),
            jax.ShapeDtypeStruct((D, 1), f32),
        ],
        interpret=_INTERPRET,
    )(rel, srel, srelsq, w0, g0c, be0c, w1)

    out = pl.pallas_call(
        functools.partial(_out_body, KP, cnt),
        grid=(B, NT),
        in_specs=[
            pl.BlockSpec((1, D, KP, TN), lambda b, n: (b, 0, 0, n)),
            pl.BlockSpec((1, D, TN), lambda b, n: (b, 0, n)),
            pl.BlockSpec((1, KP, TN, D), lambda b, n: (b * NT + n, 0, 0, 0)),
            const2((D, 1)),
            const2((D, 1)),
            const2((D, 1)),
            const2((D, 1)),
            const2((1, D)),
            const2((1, D)),
            const2((1, 1)),
        ],
        out_specs=pl.BlockSpec((1, D, TN), lambda b, n: (b, 0, n)),
        out_shape=jax.ShapeDtypeStruct((B, D, N), f32),
        interpret=_INTERPRET,
    )(y14, fg, patch, ssum1, ssq1, g1c, be1c, w2a, w2b, b2c)

    return out


# speedup vs baseline: 1.2160x; 1.2160x over previous
import functools

import jax
import jax.numpy as jnp
from jax import lax
from jax.experimental import pallas as pl
from jax.experimental.pallas import tpu as pltpu
from jax.experimental.pallas import tpu_sc as plsc

_EPS = 1e-5
_INTERPRET = False


def _dot(a, b, ca, cb, precision=None):
    return jax.lax.dot_general(
        a, b, (((ca,), (cb,)), ((), ())),
        preferred_element_type=jnp.float32, precision=precision)


def _knn_body(k, kp, orig_ref, q_ref, rel_ref, idxg_ref, srel_ref,
              srelsq_ref, m_const):
    b = pl.program_id(0)
    n = pl.program_id(1)
    orig = orig_ref[0]
    q = q_ref[0]
    tn = q.shape[1]

    cross = _dot(orig, q, 0, 0)
    q2 = jnp.sum(q * q, axis=0, keepdims=True)
    o2row = jnp.sum(orig * orig, axis=0, keepdims=True)
    o2col = jnp.swapaxes(o2row, 0, 1)
    dist = (q2 - 2.0 * cross) + o2col

    srel = jnp.zeros((3, 1), jnp.float32)
    srelsq = jnp.zeros((3, 3), jnp.float32)
    hp = jax.lax.Precision.HIGHEST

    def split3(x):
        bf = jnp.bfloat16
        hi = x.astype(bf)
        r1 = x - hi.astype(jnp.float32)
        mid = r1.astype(bf)
        lo = (r1 - mid.astype(jnp.float32)).astype(bf)
        return hi, mid, lo

    def oh_dot(parts, ohb):
        acc = None
        for p in parts:
            t = _dot(p, ohb, 1, 0)
            acc = t if acc is None else acc + t
        return acc

    orig3 = split3(orig)
    iota = jax.lax.broadcasted_iota(jnp.int32, dist.shape, 0)
    ohs = []
    idxs = []
    for j in range(k):
        m = jnp.min(dist, axis=0, keepdims=True)
        eq = dist == m
        idx = jnp.min(jnp.where(eq, iota, m_const), axis=0, keepdims=True)
        hit = iota == idx
        ohs.append(hit.astype(jnp.bfloat16))
        dist = jnp.where(hit, jnp.float32(jnp.inf), dist)
        if j < kp:
            idxs.append(idx)
            if j == kp - 1:
                idxg_ref[0, 0] = jnp.concatenate(idxs, axis=0) + b * m_const
        if (j + 1) % 4 == 0:
            ohcat = jnp.concatenate(ohs, axis=1)
            ohs = []
            ptsc = oh_dot(orig3, ohcat)
            for jj in range(4):
                jr = j - 3 + jj
                rel = ptsc[:, jj * tn:(jj + 1) * tn] - q
                rel_ref[0, jr] = rel
                srel = srel + jnp.sum(rel, axis=1, keepdims=True)
                srelsq = srelsq + _dot(rel, rel, 1, 1, precision=hp)

    first = (b == 0) & (n == 0)

    @pl.when(first)
    def _():
        srel_ref[...] = jnp.zeros_like(srel_ref)
        srelsq_ref[...] = jnp.zeros_like(srelsq_ref)

    srel_ref[...] += srel
    srelsq_ref[...] += srelsq


def _mid_body(k, kp, cnt, rel_ref, srel_ref, srelsq_ref, w0_ref, g0_ref,
              be0_ref, w1_ref, y14_ref, fg_ref, ssum1_ref, ssq1_ref):
    b = pl.program_id(0)
    n = pl.program_id(1)
    w0 = w0_ref[...]
    w1 = w1_ref[...]
    mean_rel = srel_ref[...] / cnt
    cov = (srelsq_ref[...] / cnt
           - _dot(mean_rel, mean_rel, 1, 1))
    mean0 = _dot(w0, mean_rel, 1, 0)
    wc = _dot(w0, cov, 1, 0)
    var0 = jnp.sum(wc * w0, axis=1, keepdims=True)
    scale0 = g0_ref[...] * jax.lax.rsqrt(var0 + _EPS)
    shift0 = be0_ref[...] - mean0 * scale0

    fg = None
    ssum1 = jnp.zeros_like(ssum1_ref)
    ssq1 = jnp.zeros_like(ssq1_ref)
    for j in range(k):
        y0 = _dot(w0, rel_ref[0, j], 1, 0)
        f0 = jax.nn.relu(y0 * scale0 + shift0)
        fg = f0 if fg is None else jnp.maximum(fg, f0)
        y1 = _dot(w1, f0, 1, 0)
        ssum1 = ssum1 + jnp.sum(y1, axis=1, keepdims=True)
        ssq1 = ssq1 + jnp.sum(y1 * y1, axis=1, keepdims=True)
        if j < kp:
            y14_ref[0, :, j] = y1
    fg_ref[0] = fg

    first = (b == 0) & (n == 0)

    @pl.when(first)
    def _():
        ssum1_ref[...] = jnp.zeros_like(ssum1_ref)
        ssq1_ref[...] = jnp.zeros_like(ssq1_ref)

    ssum1_ref[...] += ssum1
    ssq1_ref[...] += ssq1


def _sc_gather(table, gidx):
    tot = gidx.shape[0]
    d = table.shape[1]
    info = plsc.get_sparse_core_info()
    nc, ns = info.num_cores, info.num_subcores
    nw = nc * ns
    per = tot // nw
    ch = 128
    mesh = plsc.VectorSubcoreMesh(core_axis_name="c", subcore_axis_name="s")

    @functools.partial(
        pl.kernel, mesh=mesh,
        out_type=jax.ShapeDtypeStruct((tot, d), jnp.float32),
        scratch_types=[
            pltpu.VMEM((ch,), jnp.int32),
            pltpu.VMEM((ch, d), jnp.float32),
            pltpu.SemaphoreType.DMA,
        ],
    )
    def k(table_hbm, idx_hbm, out_hbm, idx_v, rows_v, sem):
        wid = lax.axis_index("s") * nc + lax.axis_index("c")
        base = wid * per
        for c in range(per // ch):
            off = base + c * ch
            pltpu.sync_copy(idx_hbm.at[pl.ds(off, ch)], idx_v)
            pltpu.async_copy(table_hbm.at[idx_v], rows_v, sem).wait()
            pltpu.sync_copy(rows_v, out_hbm.at[pl.ds(off, ch)])

    return k(table, gidx)


def _out_body(kp, cnt, y14_ref, fg_ref, patch_ref, ssum1_ref, ssq1_ref,
              g1_ref, be1_ref, w2a_ref, w2b_ref, b2_ref, out_ref):
    mean1 = ssum1_ref[...] / cnt
    var1 = ssq1_ref[...] / cnt - mean1 * mean1
    scale1 = g1_ref[...] * jax.lax.rsqrt(var1 + _EPS)
    shift1 = be1_ref[...] - mean1 * scale1

    fg = fg_ref[0]
    fterm = _dot(w2b_ref[...], fg, 1, 0) + b2_ref[...]
    acc = None
    for j in range(kp):
        rf = jax.nn.relu(y14_ref[0, :, j] * scale1 + shift1)
        logit = _dot(w2a_ref[...], rf, 1, 0) + fterm
        w = jax.nn.sigmoid(logit)
        pj = jnp.swapaxes(patch_ref[0, j], 0, 1)
        term = (1.0 - w) * rf + w * pj
        acc = term if acc is None else acc + term
    out_ref[0] = acc


def kernel(original_pts, query_pts, local_feat, w0, b0, g0, be0, w1, b1, g1,
           be1, w2, b2):
    del b0, b1
    B, _, M = original_pts.shape
    N = query_pts.shape[2]
    D = local_feat.shape[1]
    K = 16
    KP = K // 4
    TN = 1024 if N % 1024 == 0 else N
    NT = N // TN
    cnt = float(B * N * K)
    f32 = jnp.float32

    g0c = g0.reshape(D, 1)
    be0c = be0.reshape(D, 1)
    g1c = g1.reshape(D, 1)
    be1c = be1.reshape(D, 1)
    w2a = w2[0, :D].reshape(1, D)
    w2b = w2[0, D:].reshape(1, D)
    b2c = b2.reshape(1, 1)

    const2 = lambda shape: pl.BlockSpec(shape, lambda b, n: (0,) * len(shape))

    rel, idx4, srel, srelsq = pl.pallas_call(
        functools.partial(_knn_body, K, KP, m_const=M),
        grid=(B, NT),
        in_specs=[
            pl.BlockSpec((1, 3, M), lambda b, n: (b, 0, 0)),
            pl.BlockSpec((1, 3, TN), lambda b, n: (b, 0, n)),
        ],
        out_specs=[
            pl.BlockSpec((1, K, 3, TN), lambda b, n: (b, 0, 0, n)),
            pl.BlockSpec((1, 1, KP, TN), lambda b, n: (b, n, 0, 0)),
            const2((3, 1)),
            const2((3, 3)),
        ],
        out_shape=[
            jax.ShapeDtypeStruct((B, K, 3, N), f32),
            jax.ShapeDtypeStruct((B, NT, KP, TN), jnp.int32),
            jax.ShapeDtypeStruct((3, 1), f32),
            jax.ShapeDtypeStruct((3, 3), f32),
        ],
        interpret=_INTERPRET,
    )(original_pts, query_pts)

    table = jnp.swapaxes(local_feat, 1, 2).reshape(B * M, D)
    rows = _sc_gather(table, idx4.reshape(B * NT * KP * TN))
    patch = rows.reshape(B * NT, KP, TN, D)

    y14, fg, ssum1, ssq1 = pl.pallas_call(
        functools.partial(_mid_body, K, KP, cnt),
        grid=(B, NT),
        in_specs=[
            pl.BlockSpec((1, K, 3, TN), lambda b, n: (b, 0, 0, n)),
            const2((3, 1)),
            const2((3, 3)),
            const2((D, 3)),
            const2((D, 1)),
            const2((D, 1)),
            const2((D, D)),
        ],
        out_specs=[
            pl.BlockSpec((1, D, KP, TN), lambda b, n: (b, 0, 0, n)),
            pl.BlockSpec((1, D, TN), lambda b, n: (b, 0, n)),
            const2((D, 1)),
            const2((D, 1)),
        ],
        out_shape=[
            jax.ShapeDtypeStruct((B, D, KP, N), f32),
            jax.ShapeDtypeStruct((B, D, N), f32),
            jax.ShapeDtypeStruct((D, 1), f32),
            jax.ShapeDtypeStruct((D, 1), f32),
        ],
        interpret=_INTERPRET,
    )(rel, srel, srelsq, w0, g0c, be0c, w1)

    out = pl.pallas_call(
        functools.partial(_out_body, KP, cnt),
        grid=(B, NT),
        in_specs=[
            pl.BlockSpec((1, D, KP, TN), lambda b, n: (b, 0, 0, n)),
            pl.BlockSpec((1, D, TN), lambda b, n: (b, 0, n)),
            pl.BlockSpec((1, KP, TN, D), lambda b, n: (b * NT + n, 0, 0, 0)),
            const2((D, 1)),
            const2((D, 1)),
            const2((D, 1)),
            const2((D, 1)),
            const2((1, D)),
            const2((1, D)),
            const2((1, 1)),
        ],
        out_specs=pl.BlockSpec((1, D, TN), lambda b, n: (b, 0, n)),
        out_shape=jax.ShapeDtypeStruct((B, D, N), f32),
        interpret=_INTERPRET,
    )(y14, fg, patch, ssum1, ssq1, g1c, be1c, w2a, w2b, b2c)

    return out


# speedup vs baseline: 1.4239x; 1.1710x over previous
import functools

import jax
import jax.numpy as jnp
from jax import lax
from jax.experimental import pallas as pl
from jax.experimental.pallas import tpu as pltpu
from jax.experimental.pallas import tpu_sc as plsc

_EPS = 1e-5
_INTERPRET = False


def _dot(a, b, ca, cb, precision=None):
    return jax.lax.dot_general(
        a, b, (((ca,), (cb,)), ((), ())),
        preferred_element_type=jnp.float32, precision=precision)


def _knn_body(k, kp, orig_ref, q_ref, rel_ref, idxg_ref, srel_ref,
              srelsq_ref, m_const):
    b = pl.program_id(0)
    n = pl.program_id(1)
    orig = orig_ref[0]
    q = q_ref[0]
    tn = q.shape[1]

    cross = _dot(orig, q, 0, 0)
    q2 = jnp.sum(q * q, axis=0, keepdims=True)
    o2row = jnp.sum(orig * orig, axis=0, keepdims=True)
    o2col = jnp.swapaxes(o2row, 0, 1)
    dist = (q2 - 2.0 * cross) + o2col

    srel = jnp.zeros((3, 1), jnp.float32)
    srelsq = jnp.zeros((3, 3), jnp.float32)
    hp = jax.lax.Precision.HIGHEST

    def split3(x):
        bf = jnp.bfloat16
        hi = x.astype(bf)
        r1 = x - hi.astype(jnp.float32)
        mid = r1.astype(bf)
        lo = (r1 - mid.astype(jnp.float32)).astype(bf)
        return hi, mid, lo

    def oh_dot(parts, ohb):
        acc = None
        for p in parts:
            t = _dot(p, ohb, 1, 0)
            acc = t if acc is None else acc + t
        return acc

    orig3 = split3(orig)
    q4 = jnp.concatenate([q] * 4, axis=1)
    iota = jax.lax.broadcasted_iota(jnp.int32, dist.shape, 0)
    ohs = []
    idxs = []
    for j in range(k):
        m = jnp.min(dist, axis=0, keepdims=True)
        eq = dist == m
        idx = jnp.min(jnp.where(eq, iota, m_const), axis=0, keepdims=True)
        hit = iota == idx
        ohs.append(hit.astype(jnp.bfloat16))
        dist = jnp.where(hit, jnp.float32(jnp.inf), dist)
        if j < kp:
            idxs.append(idx)
            if j == kp - 1:
                idxg_ref[0, 0] = jnp.concatenate(idxs, axis=0) + b * m_const
        if (j + 1) % 4 == 0:
            ohcat = jnp.concatenate(ohs, axis=1)
            ohs = []
            ptsc = oh_dot(orig3, ohcat)
            rel4 = ptsc - q4
            for jj in range(4):
                rel_ref[0, j - 3 + jj] = rel4[:, jj * tn:(jj + 1) * tn]
            srel = srel + jnp.sum(rel4, axis=1, keepdims=True)
            srelsq = srelsq + _dot(rel4, rel4, 1, 1, precision=hp)

    first = (b == 0) & (n == 0)

    @pl.when(first)
    def _():
        srel_ref[...] = jnp.zeros_like(srel_ref)
        srelsq_ref[...] = jnp.zeros_like(srelsq_ref)

    srel_ref[...] += srel
    srelsq_ref[...] += srelsq


def _mid_body(k, kp, cnt, rel_ref, srel_ref, srelsq_ref, w0_ref, g0_ref,
              be0_ref, w1_ref, y14_ref, fg_ref, ssum1_ref, ssq1_ref):
    b = pl.program_id(0)
    n = pl.program_id(1)
    w0 = w0_ref[...]
    w1 = w1_ref[...]
    mean_rel = srel_ref[...] / cnt
    cov = (srelsq_ref[...] / cnt
           - _dot(mean_rel, mean_rel, 1, 1))
    mean0 = _dot(w0, mean_rel, 1, 0)
    wc = _dot(w0, cov, 1, 0)
    var0 = jnp.sum(wc * w0, axis=1, keepdims=True)
    scale0 = g0_ref[...] * jax.lax.rsqrt(var0 + _EPS)
    shift0 = be0_ref[...] - mean0 * scale0

    tn = fg_ref.shape[2]
    relcat = jnp.concatenate([rel_ref[0, j] for j in range(k)], axis=1)
    y0cat = _dot(w0, relcat, 1, 0)
    f0cat = jax.nn.relu(y0cat * scale0 + shift0)
    fg = None
    for j in range(k):
        f0 = f0cat[:, j * tn:(j + 1) * tn]
        fg = f0 if fg is None else jnp.maximum(fg, f0)
    y1cat = _dot(w1, f0cat, 1, 0)
    ssum1 = jnp.sum(y1cat, axis=1, keepdims=True)
    ssq1 = jnp.sum(y1cat * y1cat, axis=1, keepdims=True)
    for j in range(kp):
        y14_ref[0, :, j] = y1cat[:, j * tn:(j + 1) * tn]
    fg_ref[0] = fg

    first = (b == 0) & (n == 0)

    @pl.when(first)
    def _():
        ssum1_ref[...] = jnp.zeros_like(ssum1_ref)
        ssq1_ref[...] = jnp.zeros_like(ssq1_ref)

    ssum1_ref[...] += ssum1
    ssq1_ref[...] += ssq1


def _sc_gather(table, gidx):
    tot = gidx.shape[0]
    d = table.shape[1]
    info = plsc.get_sparse_core_info()
    nc, ns = info.num_cores, info.num_subcores
    nw = nc * ns
    per = tot // nw
    ch = 128
    mesh = plsc.VectorSubcoreMesh(core_axis_name="c", subcore_axis_name="s")

    @functools.partial(
        pl.kernel, mesh=mesh,
        out_type=jax.ShapeDtypeStruct((tot, d), jnp.float32),
        scratch_types=[
            pltpu.VMEM((ch,), jnp.int32),
            pltpu.VMEM((ch, d), jnp.float32),
            pltpu.SemaphoreType.DMA,
        ],
    )
    def k(table_hbm, idx_hbm, out_hbm, idx_v, rows_v, sem):
        wid = lax.axis_index("s") * nc + lax.axis_index("c")
        base = wid * per
        for c in range(per // ch):
            off = base + c * ch
            pltpu.sync_copy(idx_hbm.at[pl.ds(off, ch)], idx_v)
            pltpu.async_copy(table_hbm.at[idx_v], rows_v, sem).wait()
            pltpu.sync_copy(rows_v, out_hbm.at[pl.ds(off, ch)])

    return k(table, gidx)


def _out_body(kp, cnt, y14_ref, fg_ref, patch_ref, ssum1_ref, ssq1_ref,
              g1_ref, be1_ref, w2a_ref, w2b_ref, b2_ref, out_ref):
    mean1 = ssum1_ref[...] / cnt
    var1 = ssq1_ref[...] / cnt - mean1 * mean1
    scale1 = g1_ref[...] * jax.lax.rsqrt(var1 + _EPS)
    shift1 = be1_ref[...] - mean1 * scale1

    fg = fg_ref[0]
    fterm = _dot(w2b_ref[...], fg, 1, 0) + b2_ref[...]
    acc = None
    for j in range(kp):
        rf = jax.nn.relu(y14_ref[0, :, j] * scale1 + shift1)
        logit = _dot(w2a_ref[...], rf, 1, 0) + fterm
        w = jax.nn.sigmoid(logit)
        pj = jnp.swapaxes(patch_ref[0, j], 0, 1)
        term = (1.0 - w) * rf + w * pj
        acc = term if acc is None else acc + term
    out_ref[0] = acc


def kernel(original_pts, query_pts, local_feat, w0, b0, g0, be0, w1, b1, g1,
           be1, w2, b2):
    del b0, b1
    B, _, M = original_pts.shape
    N = query_pts.shape[2]
    D = local_feat.shape[1]
    K = 16
    KP = K // 4
    TN = 512 if N % 512 == 0 else N
    NT = N // TN
    cnt = float(B * N * K)
    f32 = jnp.float32

    g0c = g0.reshape(D, 1)
    be0c = be0.reshape(D, 1)
    g1c = g1.reshape(D, 1)
    be1c = be1.reshape(D, 1)
    w2a = w2[0, :D].reshape(1, D)
    w2b = w2[0, D:].reshape(1, D)
    b2c = b2.reshape(1, 1)

    const2 = lambda shape: pl.BlockSpec(shape, lambda b, n: (0,) * len(shape))

    rel, idx4, srel, srelsq = pl.pallas_call(
        functools.partial(_knn_body, K, KP, m_const=M),
        grid=(B, NT),
        in_specs=[
            pl.BlockSpec((1, 3, M), lambda b, n: (b, 0, 0)),
            pl.BlockSpec((1, 3, TN), lambda b, n: (b, 0, n)),
        ],
        out_specs=[
            pl.BlockSpec((1, K, 3, TN), lambda b, n: (b, 0, 0, n)),
            pl.BlockSpec((1, 1, KP, TN), lambda b, n: (b, n, 0, 0)),
            const2((3, 1)),
            const2((3, 3)),
        ],
        out_shape=[
            jax.ShapeDtypeStruct((B, K, 3, N), f32),
            jax.ShapeDtypeStruct((B, NT, KP, TN), jnp.int32),
            jax.ShapeDtypeStruct((3, 1), f32),
            jax.ShapeDtypeStruct((3, 3), f32),
        ],
        interpret=_INTERPRET,
    )(original_pts, query_pts)

    table = jnp.swapaxes(local_feat, 1, 2).reshape(B * M, D)
    rows = _sc_gather(table, idx4.reshape(B * NT * KP * TN))
    patch = rows.reshape(B * NT, KP, TN, D)

    y14, fg, ssum1, ssq1 = pl.pallas_call(
        functools.partial(_mid_body, K, KP, cnt),
        grid=(B, NT),
        in_specs=[
            pl.BlockSpec((1, K, 3, TN), lambda b, n: (b, 0, 0, n)),
            const2((3, 1)),
            const2((3, 3)),
            const2((D, 3)),
            const2((D, 1)),
            const2((D, 1)),
            const2((D, D)),
        ],
        out_specs=[
            pl.BlockSpec((1, D, KP, TN), lambda b, n: (b, 0, 0, n)),
            pl.BlockSpec((1, D, TN), lambda b, n: (b, 0, n)),
            const2((D, 1)),
            const2((D, 1)),
        ],
        out_shape=[
            jax.ShapeDtypeStruct((B, D, KP, N), f32),
            jax.ShapeDtypeStruct((B, D, N), f32),
            jax.ShapeDtypeStruct((D, 1), f32),
            jax.ShapeDtypeStruct((D, 1), f32),
        ],
        interpret=_INTERPRET,
    )(rel, srel, srelsq, w0, g0c, be0c, w1)

    out = pl.pallas_call(
        functools.partial(_out_body, KP, cnt),
        grid=(B, NT),
        in_specs=[
            pl.BlockSpec((1, D, KP, TN), lambda b, n: (b, 0, 0, n)),
            pl.BlockSpec((1, D, TN), lambda b, n: (b, 0, n)),
            pl.BlockSpec((1, KP, TN, D), lambda b, n: (b * NT + n, 0, 0, 0)),
            const2((D, 1)),
            const2((D, 1)),
            const2((D, 1)),
            const2((D, 1)),
            const2((1, D)),
            const2((1, D)),
            const2((1, 1)),
        ],
        out_specs=pl.BlockSpec((1, D, TN), lambda b, n: (b, 0, n)),
        out_shape=jax.ShapeDtypeStruct((B, D, N), f32),
        interpret=_INTERPRET,
    )(y14, fg, patch, ssum1, ssq1, g1c, be1c, w2a, w2b, b2c)

    return out


# speedup vs baseline: 1.6683x; 1.1716x over previous
import functools

import jax
import jax.numpy as jnp
from jax import lax
from jax.experimental import pallas as pl
from jax.experimental.pallas import tpu as pltpu
from jax.experimental.pallas import tpu_sc as plsc

_EPS = 1e-5
_INTERPRET = False


def _dot(a, b, ca, cb, precision=None):
    return jax.lax.dot_general(
        a, b, (((ca,), (cb,)), ((), ())),
        preferred_element_type=jnp.float32, precision=precision)


def _knn_body(k, kp, orig_ref, q_ref, rel_ref, idxg_ref, srel_ref,
              srelsq_ref, m_const):
    b = pl.program_id(0)
    n = pl.program_id(1)
    orig = orig_ref[0]
    q = q_ref[0]
    tn = q.shape[1]

    cross = _dot(orig, q, 0, 0)
    q2 = jnp.sum(q * q, axis=0, keepdims=True)
    o2row = jnp.sum(orig * orig, axis=0, keepdims=True)
    o2col = jnp.swapaxes(o2row, 0, 1)
    dist = (q2 - 2.0 * cross) + o2col

    srel = jnp.zeros((3, 1), jnp.float32)
    srelsq = jnp.zeros((3, 3), jnp.float32)
    hp = jax.lax.Precision.HIGHEST

    def split3(x):
        bf = jnp.bfloat16
        hi = x.astype(bf)
        r1 = x - hi.astype(jnp.float32)
        mid = r1.astype(bf)
        lo = (r1 - mid.astype(jnp.float32)).astype(bf)
        return hi, mid, lo

    def oh_dot(parts, ohb):
        acc = None
        for p in parts:
            t = _dot(p, ohb, 1, 0)
            acc = t if acc is None else acc + t
        return acc

    orig3 = split3(orig)
    q4 = jnp.concatenate([q] * 4, axis=1)
    iota = jax.lax.broadcasted_iota(jnp.int32, dist.shape, 0)
    ohs = []
    idxs = []
    for j in range(k):
        idx = jnp.argmin(dist, axis=0)[None, :]
        hit = iota == idx
        ohs.append(hit.astype(jnp.bfloat16))
        dist = jnp.where(hit, jnp.float32(jnp.inf), dist)
        if j < kp:
            idxs.append(idx)
            if j == kp - 1:
                idxg_ref[0, 0] = jnp.concatenate(idxs, axis=0) + b * m_const
        if (j + 1) % 4 == 0:
            ohcat = jnp.concatenate(ohs, axis=1)
            ohs = []
            ptsc = oh_dot(orig3, ohcat)
            rel4 = ptsc - q4
            for jj in range(4):
                rel_ref[0, j - 3 + jj] = rel4[:, jj * tn:(jj + 1) * tn]
            srel = srel + jnp.sum(rel4, axis=1, keepdims=True)
            srelsq = srelsq + _dot(rel4, rel4, 1, 1, precision=hp)

    first = (b == 0) & (n == 0)

    @pl.when(first)
    def _():
        srel_ref[...] = jnp.zeros_like(srel_ref)
        srelsq_ref[...] = jnp.zeros_like(srelsq_ref)

    srel_ref[...] += srel
    srelsq_ref[...] += srelsq


def _mid_body(k, kp, cnt, rel_ref, srel_ref, srelsq_ref, w0_ref, g0_ref,
              be0_ref, w1_ref, y14_ref, fg_ref, ssum1_ref, ssq1_ref):
    b = pl.program_id(0)
    n = pl.program_id(1)
    w0 = w0_ref[...]
    w1 = w1_ref[...]
    mean_rel = srel_ref[...] / cnt
    cov = (srelsq_ref[...] / cnt
           - _dot(mean_rel, mean_rel, 1, 1))
    mean0 = _dot(w0, mean_rel, 1, 0)
    wc = _dot(w0, cov, 1, 0)
    var0 = jnp.sum(wc * w0, axis=1, keepdims=True)
    scale0 = g0_ref[...] * jax.lax.rsqrt(var0 + _EPS)
    shift0 = be0_ref[...] - mean0 * scale0

    tn = fg_ref.shape[2]
    relcat = jnp.concatenate([rel_ref[0, j] for j in range(k)], axis=1)
    y0cat = _dot(w0, relcat, 1, 0)
    f0cat = jax.nn.relu(y0cat * scale0 + shift0)
    fg = None
    for j in range(k):
        f0 = f0cat[:, j * tn:(j + 1) * tn]
        fg = f0 if fg is None else jnp.maximum(fg, f0)
    y1cat = _dot(w1, f0cat, 1, 0)
    ssum1 = jnp.sum(y1cat, axis=1, keepdims=True)
    ssq1 = jnp.sum(y1cat * y1cat, axis=1, keepdims=True)
    for j in range(kp):
        y14_ref[0, :, j] = y1cat[:, j * tn:(j + 1) * tn]
    fg_ref[0] = fg

    first = (b == 0) & (n == 0)

    @pl.when(first)
    def _():
        ssum1_ref[...] = jnp.zeros_like(ssum1_ref)
        ssq1_ref[...] = jnp.zeros_like(ssq1_ref)

    ssum1_ref[...] += ssum1
    ssq1_ref[...] += ssq1


def _sc_gather(table, gidx):
    tot = gidx.shape[0]
    d = table.shape[1]
    info = plsc.get_sparse_core_info()
    nc, ns = info.num_cores, info.num_subcores
    nw = nc * ns
    per = tot // nw
    ch = 128
    mesh = plsc.VectorSubcoreMesh(core_axis_name="c", subcore_axis_name="s")

    @functools.partial(
        pl.kernel, mesh=mesh,
        out_type=jax.ShapeDtypeStruct((tot, d), jnp.float32),
        scratch_types=[
            pltpu.VMEM((ch,), jnp.int32),
            pltpu.VMEM((ch, d), jnp.float32),
            pltpu.SemaphoreType.DMA,
        ],
    )
    def k(table_hbm, idx_hbm, out_hbm, idx_v, rows_v, sem):
        wid = lax.axis_index("s") * nc + lax.axis_index("c")
        base = wid * per
        for c in range(per // ch):
            off = base + c * ch
            pltpu.sync_copy(idx_hbm.at[pl.ds(off, ch)], idx_v)
            pltpu.async_copy(table_hbm.at[idx_v], rows_v, sem).wait()
            pltpu.sync_copy(rows_v, out_hbm.at[pl.ds(off, ch)])

    return k(table, gidx)


def _out_body(kp, cnt, y14_ref, fg_ref, patch_ref, ssum1_ref, ssq1_ref,
              g1_ref, be1_ref, w2a_ref, w2b_ref, b2_ref, out_ref):
    mean1 = ssum1_ref[...] / cnt
    var1 = ssq1_ref[...] / cnt - mean1 * mean1
    scale1 = g1_ref[...] * jax.lax.rsqrt(var1 + _EPS)
    shift1 = be1_ref[...] - mean1 * scale1

    fg = fg_ref[0]
    fterm = _dot(w2b_ref[...], fg, 1, 0) + b2_ref[...]
    acc = None
    for j in range(kp):
        rf = jax.nn.relu(y14_ref[0, :, j] * scale1 + shift1)
        logit = _dot(w2a_ref[...], rf, 1, 0) + fterm
        w = jax.nn.sigmoid(logit)
        pj = jnp.swapaxes(patch_ref[0, j], 0, 1)
        term = (1.0 - w) * rf + w * pj
        acc = term if acc is None else acc + term
    out_ref[0] = acc


def kernel(original_pts, query_pts, local_feat, w0, b0, g0, be0, w1, b1, g1,
           be1, w2, b2):
    del b0, b1
    B, _, M = original_pts.shape
    N = query_pts.shape[2]
    D = local_feat.shape[1]
    K = 16
    KP = K // 4
    TN = 512 if N % 512 == 0 else N
    NT = N // TN
    cnt = float(B * N * K)
    f32 = jnp.float32

    g0c = g0.reshape(D, 1)
    be0c = be0.reshape(D, 1)
    g1c = g1.reshape(D, 1)
    be1c = be1.reshape(D, 1)
    w2a = w2[0, :D].reshape(1, D)
    w2b = w2[0, D:].reshape(1, D)
    b2c = b2.reshape(1, 1)

    const2 = lambda shape: pl.BlockSpec(shape, lambda b, n: (0,) * len(shape))

    rel, idx4, srel, srelsq = pl.pallas_call(
        functools.partial(_knn_body, K, KP, m_const=M),
        grid=(B, NT),
        in_specs=[
            pl.BlockSpec((1, 3, M), lambda b, n: (b, 0, 0)),
            pl.BlockSpec((1, 3, TN), lambda b, n: (b, 0, n)),
        ],
        out_specs=[
            pl.BlockSpec((1, K, 3, TN), lambda b, n: (b, 0, 0, n)),
            pl.BlockSpec((1, 1, KP, TN), lambda b, n: (b, n, 0, 0)),
            const2((3, 1)),
            const2((3, 3)),
        ],
        out_shape=[
            jax.ShapeDtypeStruct((B, K, 3, N), f32),
            jax.ShapeDtypeStruct((B, NT, KP, TN), jnp.int32),
            jax.ShapeDtypeStruct((3, 1), f32),
            jax.ShapeDtypeStruct((3, 3), f32),
        ],
        interpret=_INTERPRET,
    )(original_pts, query_pts)

    table = jnp.swapaxes(local_feat, 1, 2).reshape(B * M, D)
    rows = _sc_gather(table, idx4.reshape(B * NT * KP * TN))
    patch = rows.reshape(B * NT, KP, TN, D)

    y14, fg, ssum1, ssq1 = pl.pallas_call(
        functools.partial(_mid_body, K, KP, cnt),
        grid=(B, NT),
        in_specs=[
            pl.BlockSpec((1, K, 3, TN), lambda b, n: (b, 0, 0, n)),
            const2((3, 1)),
            const2((3, 3)),
            const2((D, 3)),
            const2((D, 1)),
            const2((D, 1)),
            const2((D, D)),
        ],
        out_specs=[
            pl.BlockSpec((1, D, KP, TN), lambda b, n: (b, 0, 0, n)),
            pl.BlockSpec((1, D, TN), lambda b, n: (b, 0, n)),
            const2((D, 1)),
            const2((D, 1)),
        ],
        out_shape=[
            jax.ShapeDtypeStruct((B, D, KP, N), f32),
            jax.ShapeDtypeStruct((B, D, N), f32),
            jax.ShapeDtypeStruct((D, 1), f32),
            jax.ShapeDtypeStruct((D, 1), f32),
        ],
        interpret=_INTERPRET,
    )(rel, srel, srelsq, w0, g0c, be0c, w1)

    out = pl.pallas_call(
        functools.partial(_out_body, KP, cnt),
        grid=(B, NT),
        in_specs=[
            pl.BlockSpec((1, D, KP, TN), lambda b, n: (b, 0, 0, n)),
            pl.BlockSpec((1, D, TN), lambda b, n: (b, 0, n)),
            pl.BlockSpec((1, KP, TN, D), lambda b, n: (b * NT + n, 0, 0, 0)),
            const2((D, 1)),
            const2((D, 1)),
            const2((D, 1)),
            const2((D, 1)),
            const2((1, D)),
            const2((1, D)),
            const2((1, 1)),
        ],
        out_specs=pl.BlockSpec((1, D, TN), lambda b, n: (b, 0, n)),
        out_shape=jax.ShapeDtypeStruct((B, D, N), f32),
        interpret=_INTERPRET,
    )(y14, fg, patch, ssum1, ssq1, g1c, be1c, w2a, w2b, b2c)

    return out
